# baseline (speedup 1.0000x reference)
import functools

import jax
import jax.numpy as jnp
import numpy as np
from jax import lax
from jax.experimental import pallas as pl
from jax.experimental.pallas import tpu as pltpu
from jax.experimental.pallas import tpu_sc as plsc

VOCAB = 100000
D_MODEL = 768
B = 4
S = 2048

_NC = 2
_NS = 16
_NW = _NC * _NS

_SPW = S // _NW
_LANES = 16
_VPR = D_MODEL // _LANES


def _sinusoidal_pe(max_len, d_model):
    pos = np.arange(max_len, dtype=np.float64)[:, None]
    div = np.exp(
        np.arange(0, d_model, 2, dtype=np.float64) * -(np.log(10000.0) / d_model)
    )
    pe = np.zeros((max_len, d_model), dtype=np.float64)
    pe[:, 0::2] = np.sin(pos * div)
    pe[:, 1::2] = np.cos(pos * div)
    return pe.astype(np.float32)


_PE = _sinusoidal_pe(S, D_MODEL)


def _sc_body(table_hbm, idx_hbm, pe_hbm, out_hbm, idx_v, pe_v, rows_v, sem):
    wid = lax.axis_index("s") * _NC + lax.axis_index("c")
    s0 = wid * _SPW

    pltpu.sync_copy(pe_hbm.at[pl.ds(s0, _SPW)], pe_v)

    def batch(b, _):
        pltpu.sync_copy(idx_hbm.at[b, pl.ds(s0, _SPW)], idx_v)
        pltpu.async_copy(table_hbm.at[idx_v], rows_v, sem).wait()

        def row_add(r, _):
            for j in range(_VPR):
                plsc.addupdate(
                    rows_v.at[r, pl.ds(j * _LANES, _LANES)],
                    pe_v[r, pl.ds(j * _LANES, _LANES)],
                )
            return ()

        lax.fori_loop(0, _SPW, row_add, (), unroll=False)
        pltpu.sync_copy(rows_v, out_hbm.at[b, pl.ds(s0, _SPW)])
        return ()

    lax.fori_loop(0, B, batch, (), unroll=False)


@jax.jit
def _embed(idx, table, pe):
    mesh = plsc.VectorSubcoreMesh(core_axis_name="c", subcore_axis_name="s")
    out = pl.kernel(
        _sc_body,
        out_type=jax.ShapeDtypeStruct((B, S, D_MODEL), jnp.float32),
        mesh=mesh,
        scratch_types=[
            pltpu.VMEM((_SPW,), jnp.int32),
            pltpu.VMEM((_SPW, D_MODEL), jnp.float32),
            pltpu.VMEM((_SPW, D_MODEL), jnp.float32),
            pltpu.SemaphoreType.DMA,
        ],
    )(table, idx, pe)
    return out


def kernel(x, table):
    return _embed(x.astype(jnp.int32), table, jnp.asarray(_PE))
